# baseline (speedup 1.0000x reference)
import numpy as np

import jax
import jax.numpy as jnp
from jax import lax
from jax.experimental import pallas as pl
from jax.experimental.pallas import tpu as pltpu
from jax.experimental.pallas import tpu_sc as plsc


_SZ = 20
_F = 384
_BS, _DH, _DW = 32, 32, 32
_NC, _NS = 2, 16
_LN = 16


def _interp_coeffs(out_size: int, in_size: int):
    o = np.arange(out_size, dtype=np.float64)
    s = (o + 0.5) * (float(in_size) / float(out_size)) - 0.5
    s = np.maximum(s, 0.0)
    s0 = np.floor(s)
    frac = (s - s0).astype(np.float32)
    i0 = np.clip(s0.astype(np.int64), 0, in_size - 1)
    i1 = np.clip(s0.astype(np.int64) + 1, 0, in_size - 1)
    return i0, i1, (1.0 - frac).astype(np.float32), frac


_J0, _J1, _W0, _W1 = _interp_coeffs(_DW, _SZ)


def _sc_table_body(row_hbm, col_hbm, out_hbm, col_v, blk, ya_v, yb_v):
    w = lax.axis_index("c") * _NS + lax.axis_index("s")

    t = jnp.maximum(40 * w - 12, 0)
    i0s = jnp.minimum(t // 64, _SZ - 1)
    i1s = jnp.minimum(i0s + 1, _SZ - 1)
    frac = t.astype(jnp.float32) * (1.0 / 64.0) - i0s.astype(jnp.float32)
    w1v = jnp.full((_LN,), 1.0, jnp.float32) * frac
    w0v = 1.0 - w1v

    pltpu.sync_copy(col_hbm, col_v)
    pltpu.sync_copy(row_hbm.at[i0s], ya_v)
    pltpu.sync_copy(row_hbm.at[i1s], yb_v)

    def yi_chunk(k, carry):
        a = ya_v[pl.ds(k * _LN, _LN)]
        b = yb_v[pl.ds(k * _LN, _LN)]
        y = w0v * a + w1v * b
        for x in range(_DW):
            blk[x, pl.ds(_F + k * _LN, _LN)] = y
        return carry

    lax.fori_loop(0, _F // _LN, yi_chunk, 0)

    for x in range(_DW):
        j0x, j1x = int(_J0[x]), int(_J1[x])
        w0x, w1x = float(_W0[x]), float(_W1[x])

        def xi_chunk(k, carry, j0x=j0x, j1x=j1x, w0x=w0x, w1x=w1x, x=x):
            a = col_v[j0x, pl.ds(k * _LN, _LN)]
            b = col_v[j1x, pl.ds(k * _LN, _LN)]
            blk[x, pl.ds(k * _LN, _LN)] = w0x * a + w1x * b
            return carry

        lax.fori_loop(0, _F // _LN, xi_chunk, 0)

    pltpu.sync_copy(blk, out_hbm.at[pl.ds(w * _DW, _DW)])


def _bcast_body(table_ref, out_ref):
    out_ref[...] = jnp.broadcast_to(table_ref[...][None], out_ref.shape)


def kernel(row_w, col_w, bs, dh, dw):
    del bs, dh, dw

    sc_table = pl.kernel(
        _sc_table_body,
        out_type=jax.ShapeDtypeStruct((_DH * _DW, 2 * _F), jnp.float32),
        mesh=plsc.VectorSubcoreMesh(core_axis_name="c", subcore_axis_name="s"),
        scratch_types=[
            pltpu.VMEM((_SZ, _F), jnp.float32),
            pltpu.VMEM((_DW, 2 * _F), jnp.float32),
            pltpu.VMEM((_F,), jnp.float32),
            pltpu.VMEM((_F,), jnp.float32),
        ],
        compiler_params=pltpu.CompilerParams(use_tc_tiling_on_sc=True),
    )
    tableT = sc_table(row_w, col_w)

    _BB = 4
    out = pl.pallas_call(
        _bcast_body,
        grid=(_BS // _BB,),
        in_specs=[pl.BlockSpec((_DH * _DW, 2 * _F), lambda b: (0, 0))],
        out_specs=pl.BlockSpec((_BB, _DH * _DW, 2 * _F), lambda b: (b, 0, 0)),
        out_shape=jax.ShapeDtypeStruct((_BS, _DH * _DW, 2 * _F), jnp.float32),
    )(tableT)
    return out.transpose(0, 2, 1).reshape(_BS, 2 * _F, _DH, _DW)


# speedup vs baseline: 1.1654x; 1.1654x over previous
import numpy as np

import jax
import jax.numpy as jnp
from jax import lax
from jax.experimental import pallas as pl
from jax.experimental.pallas import tpu as pltpu
from jax.experimental.pallas import tpu_sc as plsc


_SZ = 20
_F = 384
_BS, _DH, _DW = 32, 32, 32
_NC, _NS = 2, 16
_LN = 16


def _sc_interp_body(row_hbm, col_hbm, out_hbm, blk, xa_v, xb_v, ya_v, yb_v):
    w = lax.axis_index("c") * _NS + lax.axis_index("s")

    t = jnp.maximum(40 * w - 12, 0)
    i0s = jnp.minimum(t // 64, _SZ - 1)
    i1s = jnp.minimum(i0s + 1, _SZ - 1)
    frac = t.astype(jnp.float32) * (1.0 / 64.0) - i0s.astype(jnp.float32)
    w1v = jnp.full((_LN,), 1.0, jnp.float32) * frac
    w0v = 1.0 - w1v

    pltpu.sync_copy(col_hbm.at[i0s], xa_v)
    pltpu.sync_copy(col_hbm.at[i1s], xb_v)
    pltpu.sync_copy(row_hbm.at[i0s], ya_v)
    pltpu.sync_copy(row_hbm.at[i1s], yb_v)

    def chunk(k, carry):
        xv = w0v * xa_v[pl.ds(k * _LN, _LN)] + w1v * xb_v[pl.ds(k * _LN, _LN)]
        yv = w0v * ya_v[pl.ds(k * _LN, _LN)] + w1v * yb_v[pl.ds(k * _LN, _LN)]
        blk[pl.ds(k * _LN, _LN)] = xv
        blk[pl.ds(_F + k * _LN, _LN)] = yv
        return carry

    lax.fori_loop(0, _F // _LN, chunk, 0)
    pltpu.sync_copy(blk, out_hbm.at[w])


def _bcast_body(interp_ref, out_ref):
    xi = interp_ref[:, 0:_F]
    yi = interp_ref[:, _F:2 * _F]
    first = jnp.broadcast_to(xi[None, :, :], (_DH, _DW, _F)).reshape(
        _DH * _DW, _F)
    second = jnp.broadcast_to(yi[:, None, :], (_DH, _DW, _F)).reshape(
        _DH * _DW, _F)
    table = jnp.concatenate([first, second], axis=1)
    out_ref[...] = jnp.broadcast_to(table[None], out_ref.shape)


def kernel(row_w, col_w, bs, dh, dw):
    del bs, dh, dw

    sc_interp = pl.kernel(
        _sc_interp_body,
        out_type=jax.ShapeDtypeStruct((_DH, 2 * _F), jnp.float32),
        mesh=plsc.VectorSubcoreMesh(core_axis_name="c", subcore_axis_name="s"),
        scratch_types=[
            pltpu.VMEM((2 * _F,), jnp.float32),
            pltpu.VMEM((_F,), jnp.float32),
            pltpu.VMEM((_F,), jnp.float32),
            pltpu.VMEM((_F,), jnp.float32),
            pltpu.VMEM((_F,), jnp.float32),
        ],
        compiler_params=pltpu.CompilerParams(use_tc_tiling_on_sc=True),
    )
    interp = sc_interp(row_w, col_w)

    _BB = 4
    out = pl.pallas_call(
        _bcast_body,
        grid=(_BS // _BB,),
        in_specs=[pl.BlockSpec((_DH, 2 * _F), lambda b: (0, 0))],
        out_specs=pl.BlockSpec((_BB, _DH * _DW, 2 * _F), lambda b: (b, 0, 0)),
        out_shape=jax.ShapeDtypeStruct((_BS, _DH * _DW, 2 * _F), jnp.float32),
    )(interp)
    return out.transpose(0, 2, 1).reshape(_BS, 2 * _F, _DH, _DW)


# speedup vs baseline: 1.1882x; 1.0196x over previous
import numpy as np

import jax
import jax.numpy as jnp
from jax import lax
from jax.experimental import pallas as pl
from jax.experimental.pallas import tpu as pltpu
from jax.experimental.pallas import tpu_sc as plsc


_SZ = 20
_F = 384
_BS, _DH, _DW = 32, 32, 32
_NC, _NS = 2, 16
_LN = 16


def _sc_interp_body(row_hbm, col_hbm, out_hbm, blk, xa_v, xb_v, ya_v, yb_v, sem):
    w = lax.axis_index("c") * _NS + lax.axis_index("s")

    t = jnp.maximum(40 * w - 12, 0)
    i0s = jnp.minimum(t // 64, _SZ - 1)
    i1s = jnp.minimum(i0s + 1, _SZ - 1)
    frac = t.astype(jnp.float32) * (1.0 / 64.0) - i0s.astype(jnp.float32)
    w1v = jnp.full((_LN,), 1.0, jnp.float32) * frac
    w0v = 1.0 - w1v

    cps = [pltpu.async_copy(col_hbm.at[i0s], xa_v, sem),
           pltpu.async_copy(col_hbm.at[i1s], xb_v, sem),
           pltpu.async_copy(row_hbm.at[i0s], ya_v, sem),
           pltpu.async_copy(row_hbm.at[i1s], yb_v, sem)]
    for cp in cps:
        cp.wait()

    for k in range(_F // _LN):
        xv = w0v * xa_v[pl.ds(k * _LN, _LN)] + w1v * xb_v[pl.ds(k * _LN, _LN)]
        yv = w0v * ya_v[pl.ds(k * _LN, _LN)] + w1v * yb_v[pl.ds(k * _LN, _LN)]
        blk[pl.ds(k * _LN, _LN)] = xv
        blk[pl.ds(_F + k * _LN, _LN)] = yv

    pltpu.sync_copy(blk, out_hbm.at[w])


def _bcast_body(interp_ref, out_ref):
    xi = interp_ref[:, 0:_F]
    yi = interp_ref[:, _F:2 * _F]
    first = jnp.broadcast_to(xi[None, :, :], (_DH, _DW, _F)).reshape(
        _DH * _DW, _F)
    second = jnp.broadcast_to(yi[:, None, :], (_DH, _DW, _F)).reshape(
        _DH * _DW, _F)
    table = jnp.concatenate([first, second], axis=1)
    out_ref[...] = jnp.broadcast_to(table[None], out_ref.shape)


def kernel(row_w, col_w, bs, dh, dw):
    del bs, dh, dw

    sc_interp = pl.kernel(
        _sc_interp_body,
        out_type=jax.ShapeDtypeStruct((_DH, 2 * _F), jnp.float32),
        mesh=plsc.VectorSubcoreMesh(core_axis_name="c", subcore_axis_name="s"),
        scratch_types=[
            pltpu.VMEM((2 * _F,), jnp.float32),
            pltpu.VMEM((_F,), jnp.float32),
            pltpu.VMEM((_F,), jnp.float32),
            pltpu.VMEM((_F,), jnp.float32),
            pltpu.VMEM((_F,), jnp.float32),
            pltpu.SemaphoreType.DMA,
        ],
        compiler_params=pltpu.CompilerParams(use_tc_tiling_on_sc=True),
    )
    interp = sc_interp(row_w, col_w)

    _BB = 4
    out = pl.pallas_call(
        _bcast_body,
        grid=(_BS // _BB,),
        in_specs=[pl.BlockSpec((_DH, 2 * _F), lambda b: (0, 0))],
        out_specs=pl.BlockSpec((_BB, _DH * _DW, 2 * _F), lambda b: (b, 0, 0)),
        out_shape=jax.ShapeDtypeStruct((_BS, _DH * _DW, 2 * _F), jnp.float32),
    )(interp)
    return out.transpose(0, 2, 1).reshape(_BS, 2 * _F, _DH, _DW)


# speedup vs baseline: 1.1938x; 1.0047x over previous
import jax
import jax.numpy as jnp
from jax import lax
from jax.experimental import pallas as pl
from jax.experimental.pallas import tpu as pltpu
from jax.experimental.pallas import tpu_sc as plsc


_SZ = 20
_F = 384
_BS, _DH, _DW = 32, 32, 32
_NS = 16
_LN = 16


def _sc_interp_body(row_hbm, col_hbm, out_hbm, blk, xa_v, xb_v, ya_v, yb_v, sem):
    w = lax.axis_index("c") * _NS + lax.axis_index("s")

    t = jnp.maximum(40 * w - 12, 0)
    i0s = jnp.minimum(t // 64, _SZ - 1)
    i1s = jnp.minimum(i0s + 1, _SZ - 1)
    frac = t.astype(jnp.float32) * (1.0 / 64.0) - i0s.astype(jnp.float32)
    w1v = jnp.full((_LN,), 1.0, jnp.float32) * frac
    w0v = 1.0 - w1v

    cps = [pltpu.async_copy(col_hbm.at[i0s], xa_v, sem),
           pltpu.async_copy(col_hbm.at[i1s], xb_v, sem),
           pltpu.async_copy(row_hbm.at[i0s], ya_v, sem),
           pltpu.async_copy(row_hbm.at[i1s], yb_v, sem)]
    for cp in cps:
        cp.wait()

    for k in range(_F // _LN):
        xv = w0v * xa_v[pl.ds(k * _LN, _LN)] + w1v * xb_v[pl.ds(k * _LN, _LN)]
        yv = w0v * ya_v[pl.ds(k * _LN, _LN)] + w1v * yb_v[pl.ds(k * _LN, _LN)]
        blk[pl.ds(k * _LN, _LN)] = xv
        blk[pl.ds(_F + k * _LN, _LN)] = yv

    pltpu.sync_copy(blk, out_hbm.at[w])


def _bcast_body(interp_ref, out_ref):
    xi = interp_ref[:, 0:_F]
    yi = interp_ref[:, _F:2 * _F]
    first = jnp.broadcast_to(xi[None, :, :], (_DH, _DW, _F)).reshape(
        _DH * _DW, _F)
    second = jnp.broadcast_to(yi[:, None, :], (_DH, _DW, _F)).reshape(
        _DH * _DW, _F)
    table = jnp.concatenate([first, second], axis=1)
    out_ref[...] = jnp.broadcast_to(table[None], out_ref.shape)


def kernel(row_w, col_w, bs, dh, dw):
    del bs, dh, dw

    sc_interp = pl.kernel(
        _sc_interp_body,
        out_type=jax.ShapeDtypeStruct((_DH, 2 * _F), jnp.float32),
        mesh=plsc.VectorSubcoreMesh(core_axis_name="c", subcore_axis_name="s"),
        scratch_types=[
            pltpu.VMEM((2 * _F,), jnp.float32),
            pltpu.VMEM((_F,), jnp.float32),
            pltpu.VMEM((_F,), jnp.float32),
            pltpu.VMEM((_F,), jnp.float32),
            pltpu.VMEM((_F,), jnp.float32),
            pltpu.SemaphoreType.DMA,
        ],
        compiler_params=pltpu.CompilerParams(use_tc_tiling_on_sc=True),
    )
    interp = sc_interp(row_w, col_w)

    _BB = 4
    out = pl.pallas_call(
        _bcast_body,
        grid=(_BS // _BB,),
        in_specs=[pl.BlockSpec((_DH, 2 * _F), lambda b: (0, 0))],
        out_specs=pl.BlockSpec((_BB, _DH * _DW, 2 * _F), lambda b: (b, 0, 0)),
        out_shape=jax.ShapeDtypeStruct((_BS, _DH * _DW, 2 * _F), jnp.float32),
    )(interp)
    return out.transpose(0, 2, 1).reshape(_BS, 2 * _F, _DH, _DW)
